# baseline (speedup 1.0000x reference)
import functools

import jax
import jax.numpy as jnp
from jax import lax
from jax.experimental import pallas as pl
from jax.experimental.pallas import tpu as pltpu
from jax.experimental.pallas import tpu_sc as plsc

N = 10000
E = 320000
D_IN = 128
D_H = 128
D_OUT = 40
D_PAD = 128

NC = 2
NS = 16
LANES = 16
NW = NC * NS
EPW = E // NW
S = 80
G2 = EPW // S
NP = 10240
RPS = NP // NS
DEGW = 16


def _mesh():
  return plsc.VectorSubcoreMesh(core_axis_name="c", subcore_axis_name="s")


def _fill(buf, rows, width, value):
  @pl.loop(0, rows)
  def _(i):
    @pl.loop(0, width, step=LANES)
    def _(j):
      buf[i, pl.ds(j, LANES)] = jnp.full((LANES,), value, jnp.float32)


def _degrees(srcf, dstf):
  @functools.partial(
      pl.kernel,
      out_type=[
          jax.ShapeDtypeStruct((NC, NP, DEGW), jnp.float32),
          jax.ShapeDtypeStruct((NC, NP, DEGW), jnp.float32),
      ],
      mesh=_mesh(),
      scratch_types=[
          pltpu.VMEM((S,), jnp.int32),
          pltpu.VMEM((S,), jnp.int32),
          pltpu.VMEM((S,), jnp.int32),
          pltpu.VMEM((S,), jnp.int32),
          pltpu.VMEM((S, DEGW), jnp.float32),
          pltpu.VMEM((S, DEGW), jnp.float32),
          pltpu.VMEM_SHARED((NP, DEGW), jnp.float32),
          pltpu.VMEM_SHARED((NP, DEGW), jnp.float32),
          pltpu.SemaphoreType.DMA,
          pltpu.SemaphoreType.DMA,
          pltpu.SemaphoreType.DMA,
          pltpu.SemaphoreType.DMA,
          pltpu.SemaphoreType.DMA,
      ],
      compiler_params=pltpu.CompilerParams(use_tc_tiling_on_sc=False),
  )
  def deg_kernel(src_hbm, dst_hbm, dego_hbm, degi_hbm,
                 sidx0, sidx1, didx0, didx1, ones_v, zbuf,
                 acco, acci, ssem0, ssem1, zsem, isem0, isem1):
    sidx = [sidx0, sidx1]
    didx = [didx0, didx1]
    isem = [isem0, isem1]
    ssem = [ssem0, ssem1]
    cid = lax.axis_index("c")
    sid = lax.axis_index("s")
    wid = sid * NC + cid
    base = wid * EPW

    def i_copy(g, s):
      a = pltpu.make_async_copy(src_hbm.at[pl.ds(base + g * S, S)],
                                sidx[s], isem[s])
      b = pltpu.make_async_copy(dst_hbm.at[pl.ds(base + g * S, S)],
                                didx[s], isem[s])
      return a, b

    def i_start(g, s):
      a, b = i_copy(g, s)
      a.start()
      b.start()

    def i_wait(g, s):
      a, b = i_copy(g, s)
      a.wait()
      b.wait()

    def s_copies(s):
      a = pltpu.make_async_copy(ones_v, acco.at[sidx[s]], ssem[s])
      b = pltpu.make_async_copy(ones_v, acci.at[didx[s]], ssem[s])
      return a, b

    def fire_s(s):
      a, b = s_copies(s)
      a.start(add=True)
      b.start(add=True)

    def drain_s(s):
      a, b = s_copies(s)
      a.wait()
      b.wait()

    i_start(0, 0)
    _fill(ones_v, S, DEGW, 1.0)
    _fill(zbuf, S, DEGW, 0.0)

    @pl.loop(0, RPS // S)
    def _(i):
      r0 = sid * RPS + i * S
      pltpu.make_async_copy(zbuf, acco.at[pl.ds(r0, S)], zsem).start()
      pltpu.make_async_copy(zbuf, acci.at[pl.ds(r0, S)], zsem).start()

    @pl.loop(0, RPS // S)
    def _(i):
      r0 = sid * RPS + i * S
      pltpu.make_async_copy(zbuf, acco.at[pl.ds(r0, S)], zsem).wait()
      pltpu.make_async_copy(zbuf, acci.at[pl.ds(r0, S)], zsem).wait()

    i_wait(0, 0)
    plsc.subcore_barrier()
    fire_s(0)

    @pl.loop(0, G2 - 1, step=2)
    def _(g):
      @pl.when(g > 0)
      def _():
        drain_s(1)

      i_start(g + 1, 1)
      drain_s(0)
      i_start(g + 2, 0)
      i_wait(g + 1, 1)
      fire_s(1)
      i_wait(g + 2, 0)
      fire_s(0)

    drain_s(1)
    drain_s(0)

    plsc.subcore_barrier()
    r0 = sid * RPS
    pltpu.sync_copy(acco.at[pl.ds(r0, RPS)], dego_hbm.at[cid, pl.ds(r0, RPS)])
    pltpu.sync_copy(acci.at[pl.ds(r0, RPS)], degi_hbm.at[cid, pl.ds(r0, RPS)])

  return deg_kernel(srcf, dstf)


def _aggregate(h, srcf, dstf, width):
  @functools.partial(
      pl.kernel,
      out_type=jax.ShapeDtypeStruct((NC, NP, width), jnp.float32),
      mesh=_mesh(),
      scratch_types=[
          pltpu.VMEM((S,), jnp.int32),
          pltpu.VMEM((S,), jnp.int32),
          pltpu.VMEM((S,), jnp.int32),
          pltpu.VMEM((S,), jnp.int32),
          pltpu.VMEM((S, width), jnp.float32),
          pltpu.VMEM((S, width), jnp.float32),
          pltpu.VMEM_SHARED((NP, width), jnp.float32),
          pltpu.SemaphoreType.DMA,
          pltpu.SemaphoreType.DMA,
          pltpu.SemaphoreType.DMA,
          pltpu.SemaphoreType.DMA,
          pltpu.SemaphoreType.DMA,
          pltpu.SemaphoreType.DMA,
          pltpu.SemaphoreType.DMA,
      ],
      compiler_params=pltpu.CompilerParams(use_tc_tiling_on_sc=False),
  )
  def agg_kernel(h_hbm, src_hbm, dst_hbm, out_hbm,
                 sidx0, sidx1, didx0, didx1, rows0, rows1, acc,
                 gsem, ssem, zsem, sisem0, sisem1, disem0, disem1):
    sidx = [sidx0, sidx1]
    didx = [didx0, didx1]
    rows = [rows0, rows1]
    sisem = [sisem0, sisem1]
    disem = [disem0, disem1]

    cid = lax.axis_index("c")
    sid = lax.axis_index("s")
    wid = sid * NC + cid
    base = wid * EPW

    def si_copy(g, s):
      return pltpu.make_async_copy(src_hbm.at[pl.ds(base + g * S, S)],
                                   sidx[s], sisem[s])

    def di_copy(g, s):
      return pltpu.make_async_copy(dst_hbm.at[pl.ds(base + g * S, S)],
                                   didx[s], disem[s])

    NSUB = 5
    SB = S // NSUB

    def g_copies(s):
      return [
          pltpu.make_async_copy(h_hbm.at[sidx[s].at[pl.ds(b * SB, SB)]],
                                rows[s].at[pl.ds(b * SB, SB)], gsem)
          for b in range(NSUB)
      ]

    class _G:
      def __init__(self, s):
        self.cs = g_copies(s)

      def start(self):
        for c in self.cs:
          c.start()

      def wait(self):
        for c in self.cs:
          c.wait()

    def g_copy(s):
      return _G(s)

    def s_copy(s):
      return pltpu.make_async_copy(rows[s], acc.at[didx[s]], ssem)

    si_copy(0, 0).start()
    di_copy(0, 0).start()

    _fill(rows[0], S, width, 0.0)

    @pl.loop(0, RPS // S)
    def _(i):
      pltpu.make_async_copy(rows[0], acc.at[pl.ds(sid * RPS + i * S, S)],
                            zsem).start()

    @pl.loop(0, RPS // S)
    def _(i):
      pltpu.make_async_copy(rows[0], acc.at[pl.ds(sid * RPS + i * S, S)],
                            zsem).wait()

    si_copy(0, 0).wait()
    g_copy(0).start()
    plsc.subcore_barrier()

    @pl.loop(0, G2 - 1, step=2)
    def _(g):
      si_copy(g + 1, 1).start()
      g_copy(0).wait()

      @pl.when(g > 0)
      def _():
        s_copy(1).wait()

      di_copy(g + 1, 1).start()
      si_copy(g + 1, 1).wait()
      di_copy(g + 1, 1).wait()
      g_copy(1).start()
      di_copy(g, 0).wait()
      s_copy(0).start(add=True)
      si_copy(g + 2, 0).start()
      g_copy(1).wait()
      s_copy(0).wait()
      di_copy(g + 2, 0).start()
      si_copy(g + 2, 0).wait()
      g_copy(0).start()
      s_copy(1).start(add=True)

    g_copy(0).wait()
    s_copy(1).wait()
    di_copy(G2 - 1, 0).wait()
    s_copy(0).start(add=True)
    s_copy(0).wait()

    plsc.subcore_barrier()
    r0 = sid * RPS
    pltpu.sync_copy(acc.at[pl.ds(r0, RPS)], out_hbm.at[cid, pl.ds(r0, RPS)])

  return agg_kernel(h, srcf, dstf)


BR = 2000


def _prep(x, dego, degi):
  def body(x_ref, do_ref, di_ref, xs_ref, no_ref, ni_ref):
    do = do_ref[0, :, 0:1] + do_ref[1, :, 0:1]
    di = di_ref[0, :, 0:1] + di_ref[1, :, 0:1]
    no = jnp.where(do > 0, lax.rsqrt(jnp.maximum(do, 1.0)), 0.0)
    ni = jnp.where(di > 0, lax.rsqrt(jnp.maximum(di, 1.0)), 0.0)
    no_ref[...] = no
    ni_ref[...] = ni
    xs_ref[...] = x_ref[...] * no

  return pl.pallas_call(
      body,
      grid=(N // BR,),
      in_specs=[
          pl.BlockSpec((BR, D_IN), lambda i: (i, 0)),
          pl.BlockSpec((2, BR, DEGW), lambda i: (0, i, 0)),
          pl.BlockSpec((2, BR, DEGW), lambda i: (0, i, 0)),
      ],
      out_specs=[
          pl.BlockSpec((BR, D_IN), lambda i: (i, 0)),
          pl.BlockSpec((BR, 1), lambda i: (i, 0)),
          pl.BlockSpec((BR, 1), lambda i: (i, 0)),
      ],
      out_shape=[
          jax.ShapeDtypeStruct((N, D_IN), jnp.float32),
          jax.ShapeDtypeStruct((N, 1), jnp.float32),
          jax.ShapeDtypeStruct((N, 1), jnp.float32),
      ],
  )(x, dego, degi)


def _layer_mid(p, ni, no, W, b):
  def body(p_ref, ni_ref, no_ref, w_ref, b_ref, out_ref):
    agg = (p_ref[0] + p_ref[1]) * ni_ref[...]
    h = lax.dot_general(agg, w_ref[...], (((1,), (0,)), ((), ())),
                        preferred_element_type=jnp.float32,
                        precision=lax.Precision.HIGHEST)
    h = jnp.maximum(h + b_ref[...], 0.0)
    out_ref[...] = h * no_ref[...]

  return pl.pallas_call(
      body,
      grid=(N // BR,),
      in_specs=[
          pl.BlockSpec((2, BR, D_H), lambda i: (0, i, 0)),
          pl.BlockSpec((BR, 1), lambda i: (i, 0)),
          pl.BlockSpec((BR, 1), lambda i: (i, 0)),
          pl.BlockSpec((D_H, D_H), lambda i: (0, 0)),
          pl.BlockSpec((1, D_H), lambda i: (0, 0)),
      ],
      out_specs=pl.BlockSpec((BR, D_H), lambda i: (i, 0)),
      out_shape=jax.ShapeDtypeStruct((N, D_H), jnp.float32),
  )(p, ni, no, W.reshape(D_H, D_H), b.reshape(1, D_H))


def _layer_mid_fused(p, ni, no, W, b, W2p):
  def body(p_ref, ni_ref, no_ref, w_ref, b_ref, w2_ref, out_ref):
    agg = (p_ref[0] + p_ref[1]) * ni_ref[...]
    h = lax.dot_general(agg, w_ref[...], (((1,), (0,)), ((), ())),
                        preferred_element_type=jnp.float32,
                        precision=lax.Precision.HIGHEST)
    h = jnp.maximum(h + b_ref[...], 0.0)
    y = lax.dot_general(h, w2_ref[...], (((1,), (0,)), ((), ())),
                        preferred_element_type=jnp.float32,
                        precision=lax.Precision.HIGHEST)
    out_ref[...] = y * no_ref[...]

  return pl.pallas_call(
      body,
      grid=(N // BR,),
      in_specs=[
          pl.BlockSpec((2, BR, D_H), lambda i: (0, i, 0)),
          pl.BlockSpec((BR, 1), lambda i: (i, 0)),
          pl.BlockSpec((BR, 1), lambda i: (i, 0)),
          pl.BlockSpec((D_H, D_H), lambda i: (0, 0)),
          pl.BlockSpec((1, D_H), lambda i: (0, 0)),
          pl.BlockSpec((D_H, D_PAD), lambda i: (0, 0)),
      ],
      out_specs=pl.BlockSpec((BR, D_PAD), lambda i: (i, 0)),
      out_shape=jax.ShapeDtypeStruct((N, D_PAD), jnp.float32),
  )(p, ni, no, W.reshape(D_H, D_H), b.reshape(1, D_H), W2p)


def _final(p, ni, b2):
  def body(p_ref, ni_ref, b_ref, out_ref):
    agg = p_ref[0, :, :D_OUT] + p_ref[1, :, :D_OUT]
    out_ref[...] = agg * ni_ref[...] + b_ref[...]

  return pl.pallas_call(
      body,
      grid=(N // BR,),
      in_specs=[
          pl.BlockSpec((2, BR, D_PAD), lambda i: (0, i, 0)),
          pl.BlockSpec((BR, 1), lambda i: (i, 0)),
          pl.BlockSpec((1, D_OUT), lambda i: (0, 0)),
      ],
      out_specs=pl.BlockSpec((BR, D_OUT), lambda i: (i, 0)),
      out_shape=jax.ShapeDtypeStruct((N, D_OUT), jnp.float32),
  )(p, ni, b2.reshape(1, D_OUT))


def kernel(x, edge_index, W0, b0, W1, b1, W2, b2):
  srcf = edge_index[0]
  dstf = edge_index[1]

  dego, degi = _degrees(srcf, dstf)
  xs, no, ni = _prep(x, dego, degi)

  p1 = _aggregate(xs, srcf, dstf, D_IN)
  h1s = _layer_mid(p1, ni, no, W0, b0)

  W2p = jnp.pad(W2, ((0, 0), (0, D_PAD - D_OUT)))
  p2 = _aggregate(h1s, srcf, dstf, D_H)
  y2 = _layer_mid_fused(p2, ni, no, W1, b1, W2p)

  p3 = _aggregate(y2, srcf, dstf, D_PAD)
  return _final(p3, ni, b2)


# speedup vs baseline: 1.2398x; 1.2398x over previous
import functools

import jax
import jax.numpy as jnp
from jax import lax
from jax.experimental import pallas as pl
from jax.experimental.pallas import tpu as pltpu
from jax.experimental.pallas import tpu_sc as plsc

N = 10000
E = 320000
D_IN = 128
D_H = 128
D_OUT = 40
D_PAD = 128

NC = 2
NS = 16
LANES = 16
NW = NC * NS
EPW = E // NW
K = 25
NCHUNK = EPW // K
NB = 5
G = NCHUNK // NB
DNB = 5
DG = NCHUNK // DNB
NP = 10240
RPS = NP // NS
DEGW = 16


def _mesh():
  return plsc.VectorSubcoreMesh(core_axis_name="c", subcore_axis_name="s")


def _fill(buf, rows, width, value):
  @pl.loop(0, rows)
  def _(i):
    @pl.loop(0, width, step=LANES)
    def _(j):
      buf[i, pl.ds(j, LANES)] = jnp.full((LANES,), value, jnp.float32)


def _degrees(src3, dst3):
  @functools.partial(
      pl.kernel,
      out_type=[
          jax.ShapeDtypeStruct((NC, NP, DEGW), jnp.float32),
          jax.ShapeDtypeStruct((NC, NP, DEGW), jnp.float32),
      ],
      mesh=_mesh(),
      scratch_types=[
          pltpu.VMEM((NCHUNK, K), jnp.int32),
          pltpu.VMEM((NCHUNK, K), jnp.int32),
          pltpu.VMEM((K, DEGW), jnp.float32),
          pltpu.VMEM((80, DEGW), jnp.float32),
          pltpu.VMEM_SHARED((NP, DEGW), jnp.float32),
          pltpu.VMEM_SHARED((NP, DEGW), jnp.float32),
          pltpu.SemaphoreType.DMA,
      ],
      compiler_params=pltpu.CompilerParams(use_tc_tiling_on_sc=False),
  )
  def deg_kernel(src_hbm, dst_hbm, dego_hbm, degi_hbm,
                 sidx, didx, ones_v, zbuf, acco, acci, sem):
    cid = lax.axis_index("c")
    sid = lax.axis_index("s")
    wid = sid * NC + cid

    pltpu.sync_copy(src_hbm.at[wid], sidx)
    pltpu.sync_copy(dst_hbm.at[wid], didx)
    _fill(ones_v, K, DEGW, 1.0)
    _fill(zbuf, 80, DEGW, 0.0)

    @pl.loop(0, RPS // 80)
    def _(i):
      r0 = sid * RPS + i * 80
      pltpu.sync_copy(zbuf, acco.at[pl.ds(r0, 80)])
      pltpu.sync_copy(zbuf, acci.at[pl.ds(r0, 80)])

    plsc.subcore_barrier()

    def fire(g):
      for b in range(DNB):
        c = g * DNB + b
        pltpu.make_async_copy(ones_v, acco.at[sidx.at[c]], sem).start(add=True)
        pltpu.make_async_copy(ones_v, acci.at[didx.at[c]], sem).start(add=True)

    def drain(g):
      for b in range(DNB):
        c = g * DNB + b
        pltpu.make_async_copy(ones_v, acco.at[sidx.at[c]], sem).wait()
        pltpu.make_async_copy(ones_v, acci.at[didx.at[c]], sem).wait()

    fire(0)

    @pl.loop(1, DG)
    def _(g):
      fire(g)
      drain(g - 1)

    drain(DG - 1)

    plsc.subcore_barrier()
    r0 = sid * RPS
    pltpu.sync_copy(acco.at[pl.ds(r0, RPS)], dego_hbm.at[cid, pl.ds(r0, RPS)])
    pltpu.sync_copy(acci.at[pl.ds(r0, RPS)], degi_hbm.at[cid, pl.ds(r0, RPS)])

  return deg_kernel(src3, dst3)


def _aggregate(h, src4, dst3, width):
  @functools.partial(
      pl.kernel,
      out_type=jax.ShapeDtypeStruct((NC, NP, width), jnp.float32),
      mesh=_mesh(),
      scratch_types=(
          [pltpu.VMEM((NB, K), jnp.int32),
           pltpu.VMEM((NB, K), jnp.int32),
           pltpu.VMEM((NCHUNK, K), jnp.int32)]
          + [pltpu.VMEM((K, width), jnp.float32) for _ in range(2 * NB)]
          + [pltpu.VMEM_SHARED((NP, width), jnp.float32),
             pltpu.SemaphoreType.DMA,
             pltpu.SemaphoreType.DMA,
             pltpu.SemaphoreType.DMA,
             pltpu.SemaphoreType.DMA]
      ),
      compiler_params=pltpu.CompilerParams(use_tc_tiling_on_sc=False),
  )
  def agg_kernel(h_hbm, src_hbm, dst_hbm, out_hbm, *scr):
    sidx = [scr[0], scr[1]]
    didx = scr[2]
    rows = [[scr[3 + s * NB + b] for b in range(NB)] for s in range(2)]
    acc, gsem, ssem = scr[3 + 2 * NB], scr[4 + 2 * NB], scr[5 + 2 * NB]
    isem = [scr[6 + 2 * NB], scr[7 + 2 * NB]]

    cid = lax.axis_index("c")
    sid = lax.axis_index("s")
    wid = sid * NC + cid

    pltpu.make_async_copy(dst_hbm.at[wid], didx, ssem).start()

    z = rows[0][0]
    z2 = rows[0][1]
    _fill(z, K, width, 0.0)
    _fill(z2, K, width, 0.0)
    rem = RPS - (RPS // K) * K

    @pl.loop(0, RPS // K)
    def _(i):
      pltpu.make_async_copy(z, acc.at[pl.ds(sid * RPS + i * K, K)],
                            gsem).start()

    pltpu.make_async_copy(z2.at[pl.ds(0, rem)],
                          acc.at[pl.ds(sid * RPS + (RPS // K) * K, rem)],
                          gsem).start()

    @pl.loop(0, RPS // K)
    def _(i):
      pltpu.make_async_copy(z, acc.at[pl.ds(sid * RPS + i * K, K)],
                            gsem).wait()

    pltpu.make_async_copy(z2.at[pl.ds(0, rem)],
                          acc.at[pl.ds(sid * RPS + (RPS // K) * K, rem)],
                          gsem).wait()
    pltpu.make_async_copy(dst_hbm.at[wid], didx, ssem).wait()

    plsc.subcore_barrier()

    def g_copy(b, s):
      return pltpu.make_async_copy(h_hbm.at[sidx[s].at[b]], rows[s][b], gsem)

    def s_copy(c, buf):
      return pltpu.make_async_copy(buf, acc.at[didx.at[c]], ssem)

    def i_copy(g, s):
      return pltpu.make_async_copy(src_hbm.at[wid * G + g], sidx[s], isem[s])

    def fire_g(s):
      for b in range(NB):
        g_copy(b, s).start()

    def drain_g(s):
      for b in range(NB):
        g_copy(b, s).wait()

    def fire_s(g, s):
      for b in range(NB):
        s_copy(g * NB + b, rows[s][b]).start(add=True)

    def drain_s(g, s):
      for b in range(NB):
        s_copy(g * NB + b, rows[s][b]).wait()

    i_copy(0, 0).start()
    i_copy(0, 0).wait()
    fire_g(0)

    @pl.loop(0, G, step=2)
    def _(g):
      i_copy(g + 1, 1).start()
      drain_g(0)

      @pl.when(g + 2 < G)
      def _():
        i_copy(g + 2, 0).start()

      @pl.when(g > 0)
      def _():
        drain_s(g - 1, 1)

      i_copy(g + 1, 1).wait()
      fire_g(1)
      fire_s(g, 0)
      drain_g(1)
      drain_s(g, 0)

      @pl.when(g + 2 < G)
      def _():
        i_copy(g + 2, 0).wait()
        fire_g(0)

      fire_s(g + 1, 1)

    drain_s(G - 1, 1)

    plsc.subcore_barrier()
    r0 = sid * RPS
    pltpu.sync_copy(acc.at[pl.ds(r0, RPS)], out_hbm.at[cid, pl.ds(r0, RPS)])

  return agg_kernel(h, src4, dst3)


BR = 2000


def _prep(x, dego, degi):
  def body(x_ref, do_ref, di_ref, xs_ref, no_ref, ni_ref):
    do = do_ref[0, :, 0:1] + do_ref[1, :, 0:1]
    di = di_ref[0, :, 0:1] + di_ref[1, :, 0:1]
    no = jnp.where(do > 0, lax.rsqrt(jnp.maximum(do, 1.0)), 0.0)
    ni = jnp.where(di > 0, lax.rsqrt(jnp.maximum(di, 1.0)), 0.0)
    no_ref[...] = no
    ni_ref[...] = ni
    xs_ref[...] = x_ref[...] * no

  return pl.pallas_call(
      body,
      grid=(N // BR,),
      in_specs=[
          pl.BlockSpec((BR, D_IN), lambda i: (i, 0)),
          pl.BlockSpec((2, BR, DEGW), lambda i: (0, i, 0)),
          pl.BlockSpec((2, BR, DEGW), lambda i: (0, i, 0)),
      ],
      out_specs=[
          pl.BlockSpec((BR, D_IN), lambda i: (i, 0)),
          pl.BlockSpec((BR, 1), lambda i: (i, 0)),
          pl.BlockSpec((BR, 1), lambda i: (i, 0)),
      ],
      out_shape=[
          jax.ShapeDtypeStruct((N, D_IN), jnp.float32),
          jax.ShapeDtypeStruct((N, 1), jnp.float32),
          jax.ShapeDtypeStruct((N, 1), jnp.float32),
      ],
  )(x, dego, degi)


def _layer_mid(p, ni, no, W, b):
  def body(p_ref, ni_ref, no_ref, w_ref, b_ref, out_ref):
    agg = (p_ref[0] + p_ref[1]) * ni_ref[...]
    h = lax.dot_general(agg, w_ref[...], (((1,), (0,)), ((), ())),
                        preferred_element_type=jnp.float32,
                        precision=lax.Precision.HIGHEST)
    h = jnp.maximum(h + b_ref[...], 0.0)
    out_ref[...] = h * no_ref[...]

  return pl.pallas_call(
      body,
      grid=(N // BR,),
      in_specs=[
          pl.BlockSpec((2, BR, D_H), lambda i: (0, i, 0)),
          pl.BlockSpec((BR, 1), lambda i: (i, 0)),
          pl.BlockSpec((BR, 1), lambda i: (i, 0)),
          pl.BlockSpec((D_H, D_H), lambda i: (0, 0)),
          pl.BlockSpec((1, D_H), lambda i: (0, 0)),
      ],
      out_specs=pl.BlockSpec((BR, D_H), lambda i: (i, 0)),
      out_shape=jax.ShapeDtypeStruct((N, D_H), jnp.float32),
  )(p, ni, no, W.reshape(D_H, D_H), b.reshape(1, D_H))


def _layer_mid_fused(p, ni, no, W, b, W2p):
  def body(p_ref, ni_ref, no_ref, w_ref, b_ref, w2_ref, out_ref):
    agg = (p_ref[0] + p_ref[1]) * ni_ref[...]
    h = lax.dot_general(agg, w_ref[...], (((1,), (0,)), ((), ())),
                        preferred_element_type=jnp.float32,
                        precision=lax.Precision.HIGHEST)
    h = jnp.maximum(h + b_ref[...], 0.0)
    y = lax.dot_general(h, w2_ref[...], (((1,), (0,)), ((), ())),
                        preferred_element_type=jnp.float32,
                        precision=lax.Precision.HIGHEST)
    out_ref[...] = y * no_ref[...]

  return pl.pallas_call(
      body,
      grid=(N // BR,),
      in_specs=[
          pl.BlockSpec((2, BR, D_H), lambda i: (0, i, 0)),
          pl.BlockSpec((BR, 1), lambda i: (i, 0)),
          pl.BlockSpec((BR, 1), lambda i: (i, 0)),
          pl.BlockSpec((D_H, D_H), lambda i: (0, 0)),
          pl.BlockSpec((1, D_H), lambda i: (0, 0)),
          pl.BlockSpec((D_H, D_PAD), lambda i: (0, 0)),
      ],
      out_specs=pl.BlockSpec((BR, D_PAD), lambda i: (i, 0)),
      out_shape=jax.ShapeDtypeStruct((N, D_PAD), jnp.float32),
  )(p, ni, no, W.reshape(D_H, D_H), b.reshape(1, D_H), W2p)


def _final(p, ni, b2):
  def body(p_ref, ni_ref, b_ref, out_ref):
    agg = p_ref[0, :, :D_OUT] + p_ref[1, :, :D_OUT]
    out_ref[...] = agg * ni_ref[...] + b_ref[...]

  return pl.pallas_call(
      body,
      grid=(N // BR,),
      in_specs=[
          pl.BlockSpec((2, BR, D_PAD), lambda i: (0, i, 0)),
          pl.BlockSpec((BR, 1), lambda i: (i, 0)),
          pl.BlockSpec((1, D_OUT), lambda i: (0, 0)),
      ],
      out_specs=pl.BlockSpec((BR, D_OUT), lambda i: (i, 0)),
      out_shape=jax.ShapeDtypeStruct((N, D_OUT), jnp.float32),
  )(p, ni, b2.reshape(1, D_OUT))


def kernel(x, edge_index, W0, b0, W1, b1, W2, b2):
  src3 = edge_index[0].reshape(NW, NCHUNK, K)
  dst3 = edge_index[1].reshape(NW, NCHUNK, K)
  src4 = edge_index[0].reshape(NW * G, NB, K)

  dego, degi = _degrees(src3, dst3)
  xs, no, ni = _prep(x, dego, degi)

  p1 = _aggregate(xs, src4, dst3, D_IN)
  h1s = _layer_mid(p1, ni, no, W0, b0)

  W2p = jnp.pad(W2, ((0, 0), (0, D_PAD - D_OUT)))
  p2 = _aggregate(h1s, src4, dst3, D_H)
  y2 = _layer_mid_fused(p2, ni, no, W1, b1, W2p)

  p3 = _aggregate(y2, src4, dst3, D_PAD)
  return _final(p3, ni, b2)
